# baseline (speedup 1.0000x reference)
import functools

import jax
import jax.numpy as jnp
from jax.experimental import pallas as pl
from jax.experimental.pallas import tpu as pltpu
from jax.experimental.pallas import tpu_sc as plsc

B = 1024
L = 200
D = 64
DP = 2 * D
H = 128
POOL = 8
DENSE_DIM = 256
HEAD_H = 256
MLP_H = 128
BB = 32
GW = 128

NC = 2
NS = 16
NW = NC * NS
NTOK = B * L
NPT = NTOK // NW
NPP = (B * POOL) // NW

_SP_NAMES = ('user_tokens', 'context_tokens', 'candidate_post_tokens',
             'candidate_author_tokens', 'candidate_tokens')


def _repack_kernel(cb, a_ref, b_ref, out_ref):
    for g in range(8):
        out_ref[g * cb:(g + 1) * cb, 0:D] = a_ref[:, g, :].T
        out_ref[g * cb:(g + 1) * cb, D:DP] = b_ref[:, g, :].T


def _make_pairs(tab, cb):
    v = tab.shape[0]
    v2 = v // 2
    chunk = 8 * cb
    nblk = v2 // chunk
    view3 = tab.T.reshape(D, v // cb, cb)
    return pl.pallas_call(
        functools.partial(_repack_kernel, cb),
        grid=(nblk,),
        in_specs=[pl.BlockSpec((D, 8, cb), lambda i: (0, i, 0)),
                  pl.BlockSpec((D, 8, cb), lambda i, n=nblk: (0, i + n, 0))],
        out_specs=pl.BlockSpec((chunk, DP), lambda i: (i, 0)),
        out_shape=jax.ShapeDtypeStruct((v2, DP), jnp.float32),
    )(view3, view3)


def _sc_gather(token_pairs, tok_idx3, sp_pairs, sp_idx):
    mesh = plsc.VectorSubcoreMesh(core_axis_name="c", subcore_axis_name="s")
    out_type = (tuple(jax.ShapeDtypeStruct((NTOK, DP), jnp.float32)
                      for _ in range(3))
                + tuple(jax.ShapeDtypeStruct((B * POOL, DP), jnp.float32)
                        for _ in sp_pairs))

    @functools.partial(
        pl.kernel, out_type=out_type, mesh=mesh,
        scratch_types=[
            pltpu.VMEM((2, GW), jnp.int32),
            pltpu.VMEM((2, GW, DP), jnp.float32),
            pltpu.SemaphoreType.DMA,
            pltpu.SemaphoreType.DMA,
            pltpu.SemaphoreType.DMA,
            pltpu.SemaphoreType.DMA,
            pltpu.SemaphoreType.DMA,
        ])
    def k(tok_tab, i0, i1, i2, t0, t1, t2, t3, t4, j0, j1, j2, j3, j4,
          e0, e1, e2, o0, o1, o2, o3, o4,
          idx_v, rows_v, sem_g, sem_o0, sem_o1, sem_i0, sem_i1):
        wid = jax.lax.axis_index("s") * NC + jax.lax.axis_index("c")
        sems_o = (sem_o0, sem_o1)
        sems_i = (sem_i0, sem_i1)

        def run(tab, idx_hbm, out_hbm, n_per_tile):
            base0 = wid * n_per_tile
            g_steps = n_per_tile // GW
            for b in range(min(2, g_steps)):
                pltpu.make_async_copy(
                    idx_hbm.at[pl.ds(base0 + b * GW, GW)], idx_v.at[b],
                    sems_i[b]).start()

            @pl.loop(0, g_steps, step=2)
            def _(g):
                for b in range(2):
                    gb = g + b
                    row0 = base0 + gb * GW
                    pltpu.make_async_copy(
                        idx_hbm.at[pl.ds(0, GW)], idx_v.at[b],
                        sems_i[b]).wait()

                    @pl.when(gb >= 2)
                    def _w():
                        pltpu.make_async_copy(
                            rows_v.at[b], out_hbm.at[pl.ds(0, GW)],
                            sems_o[b]).wait()

                    pltpu.async_copy(tab.at[idx_v.at[b]], rows_v.at[b],
                                     sem_g).wait()
                    pltpu.make_async_copy(
                        rows_v.at[b], out_hbm.at[pl.ds(row0, GW)],
                        sems_o[b]).start()

                    @pl.when(gb + 2 < g_steps)
                    def _p():
                        pltpu.make_async_copy(
                            idx_hbm.at[pl.ds(row0 + 2 * GW, GW)],
                            idx_v.at[b], sems_i[b]).start()

            for b in range(min(2, g_steps)):
                pltpu.make_async_copy(
                    rows_v.at[b], out_hbm.at[pl.ds(0, GW)], sems_o[b]).wait()

        run(tok_tab, i0, e0, NPT)
        run(tok_tab, i1, e1, NPT)
        run(tok_tab, i2, e2, NPT)
        for tab, jj, oo in ((t0, j0, o0), (t1, j1, o1), (t2, j2, o2),
                            (t3, j3, o3), (t4, j4, o4)):
            run(tab, jj, oo, NPP)

    return k(token_pairs, *tok_idx3, *sp_pairs, *sp_idx)


def _dense_kernel(e0_ref, e1_ref, e2_ref, i0_ref, i1_ref, i2_ref, len_ref,
                  pu_ref, pc_ref, pcp_ref, pca_ref, pct_ref,
                  ju_ref, jc_ref, jcp_ref, jca_ref, jct_ref, dn_ref,
                  whist_ref, lnh_ref, fuser_ref, lnu_ref, fcand_ref, lnc_ref,
                  wdense_ref, lnd_ref, w1ac_ref, w1h_ref, w1p_ref, attp_ref,
                  hw1_ref, hb1_ref, hw2_ref, hb2_ref, hw3_ref, scal_ref,
                  out_ref):
    f32 = jnp.float32

    def dot(a, b):
        return jax.lax.dot_general(a, b, (((1,), (0,)), ((), ())),
                                   preferred_element_type=f32)

    def ln_act(y, lnref):
        y = y + lnref[0:1, :]
        m = jnp.mean(y, axis=-1, keepdims=True)
        v = jnp.mean((y - m) ** 2, axis=-1, keepdims=True)
        y = (y - m) * jax.lax.rsqrt(v + 1e-5) * lnref[1:2, :] + lnref[2:3, :]
        return jax.nn.gelu(y)

    def masked_pairs(e_ref, idx_ref, rows, v2):
        e3 = e_ref[...].reshape(BB, rows, DP)
        q3 = (idx_ref[...] >= v2).astype(f32)[:, :, None]
        lane = jax.lax.broadcasted_iota(jnp.int32, (BB, rows, DP), 2)
        return e3 * jnp.where(lane < D, 1.0 - q3, q3)

    whist = whist_ref[...]
    vt2 = 500000
    vs2 = 50000
    hpre = dot(masked_pairs(e0_ref, i0_ref, L, vt2).reshape(BB * L, DP),
               whist[0:DP])
    hpre += dot(masked_pairs(e1_ref, i1_ref, L, vt2).reshape(BB * L, DP),
                whist[DP:2 * DP])
    hpre += dot(masked_pairs(e2_ref, i2_ref, L, vt2).reshape(BB * L, DP),
                whist[2 * DP:3 * DP])
    hist2 = ln_act(hpre, lnh_ref)
    lens = len_ref[...]
    iota3 = jax.lax.broadcasted_iota(jnp.int32, (BB, L, 1), 1)
    mask3 = (iota3 < lens[:, :, None]).astype(f32)
    h3 = hist2.reshape(BB, L, H) * mask3
    hist2m = h3.reshape(BB * L, H)

    def pooled(ref, jref):
        return jnp.sum(masked_pairs(ref, jref, POOL, vs2), axis=1)

    p_u = pooled(pu_ref, ju_ref)
    p_c = pooled(pc_ref, jc_ref)
    p_cp = pooled(pcp_ref, jcp_ref)
    p_ca = pooled(pca_ref, jca_ref)
    p_ct = pooled(pct_ref, jct_ref)

    fuser = fuser_ref[...]
    user = ln_act(dot(p_u, fuser[0:DP]) + dot(p_c, fuser[DP:2 * DP]), lnu_ref)
    fcand = fcand_ref[...]
    cand = ln_act(dot(p_cp, fcand[0:DP]) + dot(p_ca, fcand[DP:2 * DP])
                  + dot(p_ct, fcand[2 * DP:3 * DP]), lnc_ref)
    dense = ln_act(dot(dn_ref[...], wdense_ref[...]), lnd_ref)

    att_a = scal_ref[0:1, 0:1]
    att_b2 = scal_ref[0:1, 1:2]
    a_row = dot(cand, w1ac_ref[...])
    hterm = dot(hist2m, w1h_ref[...])
    prod = h3 * cand[:, None, :]
    pterm = dot(prod.reshape(BB * L, H), w1p_ref[...])
    pre3 = ((hterm + pterm).reshape(BB, L, H) + a_row[:, None, :]
            + attp_ref[0:1, :][None])
    sact = jnp.where(pre3 >= 0, pre3, pre3 * att_a[:, :, None])
    s = jnp.sum(sact * attp_ref[1:2, :][None], axis=-1) + att_b2

    iota2 = jax.lax.broadcasted_iota(jnp.int32, (BB, L), 1)
    mask2 = iota2 < lens
    s = jnp.where(mask2, s, -1e9)
    smax = jnp.max(s, axis=-1, keepdims=True)
    e = jnp.exp(s - smax)
    w = e / jnp.sum(e, axis=-1, keepdims=True)
    context = jnp.sum(w[:, :, None] * h3, axis=1)
    denom = jnp.maximum(jnp.sum(mask2.astype(f32), axis=-1, keepdims=True),
                        1.0)
    summary = jnp.sum(h3, axis=1) / denom

    a1 = scal_ref[0:1, 2:3]
    a2 = scal_ref[0:1, 3:4]
    b3 = scal_ref[0:1, 4:5]
    hw1 = hw1_ref[...]
    pieces = (cand, context, summary, user, dense, cand * context,
              jnp.abs(cand - user))
    acc = dot(pieces[0], hw1[0:H])
    for kk in range(1, 7):
        acc = acc + dot(pieces[kk], hw1[kk * H:(kk + 1) * H])
    h1 = acc + hb1_ref[...]
    h1 = jnp.where(h1 >= 0, h1, h1 * a1)
    h2 = dot(h1, hw2_ref[...]) + hb2_ref[...]
    h2 = jnp.where(h2 >= 0, h2, h2 * a2)
    out_ref[...] = jnp.sum(h2 * hw3_ref[...], axis=-1, keepdims=True) + b3


def _fold_weights(p):
    tok_w = p['tok_W']
    tok_b = p['tok_b']

    def pack_ln(bias, g, beta):
        z = jnp.zeros((8, H), jnp.float32)
        return z.at[0].set(bias).at[1].set(g).at[2].set(beta)

    def fold(w_big, n):
        blocks = [jnp.tile(tok_w @ w_big[k * H:(k + 1) * H], (2, 1))
                  for k in range(n)]
        bias = sum(tok_b @ w_big[k * H:(k + 1) * H] for k in range(n))
        return jnp.concatenate(blocks, axis=0), bias

    whist, bh = fold(p['hist_W'], 3)
    fuser, bu = fold(p['user_W'], 2)
    fcand, bc = fold(p['cand_W'], 3)

    w1 = p['att_W1']
    w1ac = w1[0:H] + w1[2 * H:3 * H]
    w1h = w1[H:2 * H] - w1[2 * H:3 * H]
    w1p = w1[3 * H:4 * H]
    attp = jnp.zeros((8, H), jnp.float32)
    attp = attp.at[0].set(p['att_b1']).at[1].set(p['att_W2'].reshape(H))
    scal = jnp.concatenate([
        jnp.stack([p['att_a'], p['att_b2'][0], p['head_a1'], p['head_a2'],
                   p['head_b3'][0]]),
        jnp.zeros((3,), jnp.float32)]).reshape(1, 8)

    return dict(
        whist=whist,
        lnh=pack_ln(p['hist_b'] + bh, p['hist_g'], p['hist_beta']),
        fuser=fuser,
        lnu=pack_ln(p['user_b'] + bu, p['user_g'], p['user_beta']),
        fcand=fcand,
        lnc=pack_ln(p['cand_b'] + bc, p['cand_g'], p['cand_beta']),
        wdense=p['dense_W'],
        lnd=pack_ln(p['dense_b'], p['dense_g'], p['dense_beta']),
        w1ac=w1ac, w1h=w1h, w1p=w1p, attp=attp,
        hw1=p['head_W1'], hb1=p['head_b1'].reshape(1, HEAD_H),
        hw2=p['head_W2'], hb2=p['head_b2'].reshape(1, MLP_H),
        hw3=p['head_W3'].reshape(1, MLP_H),
        scal=scal,
    )


def _dense_forward(embs, tok_idx2d, lengths, pooled, sp_idx2d,
                   dense_features, p):
    fw = _fold_weights(p)
    lens2 = lengths.astype(jnp.int32).reshape(B, 1)

    def full(shp):
        return pl.BlockSpec(shp, lambda i: tuple(0 for _ in shp))

    out = pl.pallas_call(
        _dense_kernel,
        grid=(B // BB,),
        in_specs=[
            pl.BlockSpec((BB * L, DP), lambda i: (i, 0)),
            pl.BlockSpec((BB * L, DP), lambda i: (i, 0)),
            pl.BlockSpec((BB * L, DP), lambda i: (i, 0)),
            pl.BlockSpec((BB, L), lambda i: (i, 0)),
            pl.BlockSpec((BB, L), lambda i: (i, 0)),
            pl.BlockSpec((BB, L), lambda i: (i, 0)),
            pl.BlockSpec((BB, 1), lambda i: (i, 0)),
            pl.BlockSpec((BB * POOL, DP), lambda i: (i, 0)),
            pl.BlockSpec((BB * POOL, DP), lambda i: (i, 0)),
            pl.BlockSpec((BB * POOL, DP), lambda i: (i, 0)),
            pl.BlockSpec((BB * POOL, DP), lambda i: (i, 0)),
            pl.BlockSpec((BB * POOL, DP), lambda i: (i, 0)),
            pl.BlockSpec((BB, POOL), lambda i: (i, 0)),
            pl.BlockSpec((BB, POOL), lambda i: (i, 0)),
            pl.BlockSpec((BB, POOL), lambda i: (i, 0)),
            pl.BlockSpec((BB, POOL), lambda i: (i, 0)),
            pl.BlockSpec((BB, POOL), lambda i: (i, 0)),
            pl.BlockSpec((BB, DENSE_DIM), lambda i: (i, 0)),
            full((3 * DP, H)), full((8, H)), full((2 * DP, H)), full((8, H)),
            full((3 * DP, H)), full((8, H)), full((DENSE_DIM, H)),
            full((8, H)), full((H, H)), full((H, H)), full((H, H)),
            full((8, H)), full((7 * H, HEAD_H)), full((1, HEAD_H)),
            full((HEAD_H, MLP_H)), full((1, MLP_H)), full((1, MLP_H)),
            full((1, 8)),
        ],
        out_specs=pl.BlockSpec((BB, 1), lambda i: (i, 0)),
        out_shape=jax.ShapeDtypeStruct((B, 1), jnp.float32),
    )(*embs, *tok_idx2d, lens2, *pooled, *sp_idx2d, dense_features,
      fw['whist'], fw['lnh'], fw['fuser'], fw['lnu'], fw['fcand'], fw['lnc'],
      fw['wdense'], fw['lnd'], fw['w1ac'], fw['w1h'], fw['w1p'], fw['attp'],
      fw['hw1'], fw['hb1'], fw['hw2'], fw['hb2'], fw['hw3'], fw['scal'])
    return out.reshape(B)


def kernel(history_post_tokens, history_author_tokens, history_action_tokens,
           history_lengths, user_tokens_idx, context_tokens_idx,
           candidate_tokens_idx, candidate_post_tokens_idx,
           candidate_author_tokens_idx, dense_features, params):
    p = params
    tok_idx2d = [t.astype(jnp.int32)
                 for t in (history_post_tokens, history_author_tokens,
                           history_action_tokens)]
    tok_half = [jnp.where(t >= 500000, t - 500000, t).reshape(B * L)
                for t in tok_idx2d]
    sp_idx_map = {
        'user_tokens': user_tokens_idx,
        'context_tokens': context_tokens_idx,
        'candidate_post_tokens': candidate_post_tokens_idx,
        'candidate_author_tokens': candidate_author_tokens_idx,
        'candidate_tokens': candidate_tokens_idx,
    }
    sp_idx2d = [sp_idx_map[n].astype(jnp.int32) for n in _SP_NAMES]
    sp_half = [jnp.where(t >= 50000, t - 50000, t).reshape(B * POOL)
               for t in sp_idx2d]

    tok_pairs = _make_pairs(p['token_table'], 2500)
    sp_pairs = [_make_pairs(p[n + '_table'], 1250) for n in _SP_NAMES]

    gathered = _sc_gather(tok_pairs, tok_half, sp_pairs, sp_half)
    embs = gathered[0:3]
    pooled = gathered[3:]
    return _dense_forward(embs, tok_idx2d, history_lengths, pooled, sp_idx2d,
                          dense_features, p)


# speedup vs baseline: 1.1482x; 1.1482x over previous
import functools

import jax
import jax.numpy as jnp
from jax.experimental import pallas as pl
from jax.experimental.pallas import tpu as pltpu
from jax.experimental.pallas import tpu_sc as plsc

B = 1024
L = 200
D = 64
DP = 2 * D
H = 128
POOL = 8
DENSE_DIM = 256
HEAD_H = 256
MLP_H = 128
BB = 32
GW = 128

NC = 2
NS = 16
NW = NC * NS
NTOK = B * L
NPT = NTOK // NW
NPP = (B * POOL) // NW

_SP_NAMES = ('user_tokens', 'context_tokens', 'candidate_post_tokens',
             'candidate_author_tokens', 'candidate_tokens')


def _sc_gather(token_pairs, tok_idx3, sp_pairs, sp_idx):
    mesh = plsc.VectorSubcoreMesh(core_axis_name="c", subcore_axis_name="s")
    out_type = (tuple(jax.ShapeDtypeStruct((NTOK, DP), jnp.float32)
                      for _ in range(3))
                + tuple(jax.ShapeDtypeStruct((B * POOL, DP), jnp.float32)
                        for _ in sp_pairs))

    @functools.partial(
        pl.kernel, out_type=out_type, mesh=mesh,
        scratch_types=[
            pltpu.VMEM((2, GW), jnp.int32),
            pltpu.VMEM((2, GW, DP), jnp.float32),
            pltpu.SemaphoreType.DMA,
            pltpu.SemaphoreType.DMA,
            pltpu.SemaphoreType.DMA,
            pltpu.SemaphoreType.DMA,
            pltpu.SemaphoreType.DMA,
        ])
    def k(tok_tab, i0, i1, i2, t0, t1, t2, t3, t4, j0, j1, j2, j3, j4,
          e0, e1, e2, o0, o1, o2, o3, o4,
          idx_v, rows_v, sem_g, sem_o0, sem_o1, sem_i0, sem_i1):
        wid = jax.lax.axis_index("s") * NC + jax.lax.axis_index("c")
        sems_o = (sem_o0, sem_o1)
        sems_i = (sem_i0, sem_i1)

        def run(tab, idx_hbm, out_hbm, n_per_tile):
            base0 = wid * n_per_tile
            g_steps = n_per_tile // GW
            for b in range(min(2, g_steps)):
                pltpu.make_async_copy(
                    idx_hbm.at[pl.ds(base0 + b * GW, GW)], idx_v.at[b],
                    sems_i[b]).start()

            @pl.loop(0, g_steps, step=2)
            def _(g):
                for b in range(2):
                    gb = g + b
                    row0 = base0 + gb * GW
                    pltpu.make_async_copy(
                        idx_hbm.at[pl.ds(0, GW)], idx_v.at[b],
                        sems_i[b]).wait()

                    @pl.when(gb >= 2)
                    def _w():
                        pltpu.make_async_copy(
                            rows_v.at[b], out_hbm.at[pl.ds(0, GW)],
                            sems_o[b]).wait()

                    pltpu.async_copy(tab.at[idx_v.at[b]], rows_v.at[b],
                                     sem_g).wait()
                    pltpu.make_async_copy(
                        rows_v.at[b], out_hbm.at[pl.ds(row0, GW)],
                        sems_o[b]).start()

                    @pl.when(gb + 2 < g_steps)
                    def _p():
                        pltpu.make_async_copy(
                            idx_hbm.at[pl.ds(row0 + 2 * GW, GW)],
                            idx_v.at[b], sems_i[b]).start()

            for b in range(min(2, g_steps)):
                pltpu.make_async_copy(
                    rows_v.at[b], out_hbm.at[pl.ds(0, GW)], sems_o[b]).wait()

        run(tok_tab, i0, e0, NPT)
        run(tok_tab, i1, e1, NPT)
        run(tok_tab, i2, e2, NPT)
        for tab, jj, oo in ((t0, j0, o0), (t1, j1, o1), (t2, j2, o2),
                            (t3, j3, o3), (t4, j4, o4)):
            run(tab, jj, oo, NPP)

    return k(token_pairs, *tok_idx3, *sp_pairs, *sp_idx)


def _dense_kernel(e0_ref, e1_ref, e2_ref, i0_ref, i1_ref, i2_ref, len_ref,
                  pu_ref, pc_ref, pcp_ref, pca_ref, pct_ref,
                  ju_ref, jc_ref, jcp_ref, jca_ref, jct_ref, dn_ref,
                  whist_ref, lnh_ref, fuser_ref, lnu_ref, fcand_ref, lnc_ref,
                  wdense_ref, lnd_ref, w1ac_ref, w1h_ref, w1p_ref, attp_ref,
                  hw1_ref, hb1_ref, hw2_ref, hb2_ref, hw3_ref, scal_ref,
                  out_ref):
    f32 = jnp.float32

    def dot(a, b):
        return jax.lax.dot_general(a, b, (((1,), (0,)), ((), ())),
                                   preferred_element_type=f32)

    def ln_act(y, lnref):
        y = y + lnref[0:1, :]
        m = jnp.mean(y, axis=-1, keepdims=True)
        v = jnp.mean((y - m) ** 2, axis=-1, keepdims=True)
        y = (y - m) * jax.lax.rsqrt(v + 1e-5) * lnref[1:2, :] + lnref[2:3, :]
        return jax.nn.gelu(y)

    def masked_pairs(e_ref, idx_ref, rows):
        e3 = e_ref[...].reshape(BB, rows, DP)
        q3 = jnp.bitwise_and(idx_ref[...], 1).astype(f32)[:, :, None]
        lane = jax.lax.broadcasted_iota(jnp.int32, (BB, rows, DP), 2)
        return e3 * jnp.where(lane < D, 1.0 - q3, q3)

    whist = whist_ref[...]
    bf16 = jnp.bfloat16
    hpre = dot(masked_pairs(e0_ref, i0_ref, L).reshape(BB * L, DP)
               .astype(bf16), whist[0:DP])
    hpre += dot(masked_pairs(e1_ref, i1_ref, L).reshape(BB * L, DP)
                .astype(bf16), whist[DP:2 * DP])
    hpre += dot(masked_pairs(e2_ref, i2_ref, L).reshape(BB * L, DP)
                .astype(bf16), whist[2 * DP:3 * DP])
    hist2 = ln_act(hpre, lnh_ref)
    lens = len_ref[...]
    iota3 = jax.lax.broadcasted_iota(jnp.int32, (BB, L, 1), 1)
    mask3 = (iota3 < lens[:, :, None]).astype(f32)
    h3 = hist2.reshape(BB, L, H) * mask3
    hist2m = h3.reshape(BB * L, H)

    def pooled(ref, jref):
        return jnp.sum(masked_pairs(ref, jref, POOL), axis=1)

    p_u = pooled(pu_ref, ju_ref)
    p_c = pooled(pc_ref, jc_ref)
    p_cp = pooled(pcp_ref, jcp_ref)
    p_ca = pooled(pca_ref, jca_ref)
    p_ct = pooled(pct_ref, jct_ref)

    fuser = fuser_ref[...]
    user = ln_act(dot(p_u, fuser[0:DP]) + dot(p_c, fuser[DP:2 * DP]), lnu_ref)
    fcand = fcand_ref[...]
    cand = ln_act(dot(p_cp, fcand[0:DP]) + dot(p_ca, fcand[DP:2 * DP])
                  + dot(p_ct, fcand[2 * DP:3 * DP]), lnc_ref)
    dense = ln_act(dot(dn_ref[...], wdense_ref[...]), lnd_ref)

    att_a = scal_ref[0:1, 0:1]
    att_b2 = scal_ref[0:1, 1:2]
    a_row = dot(cand, w1ac_ref[...])
    hist16 = hist2m.astype(bf16)
    hterm = dot(hist16, w1h_ref[...])
    prod = h3 * cand[:, None, :]
    pterm = dot(prod.reshape(BB * L, H).astype(bf16), w1p_ref[...])
    pre3 = ((hterm + pterm).reshape(BB, L, H) + a_row[:, None, :]
            + attp_ref[0:1, :][None])
    sact = jnp.where(pre3 >= 0, pre3, pre3 * att_a[:, :, None])
    s = jnp.sum(sact * attp_ref[1:2, :][None], axis=-1) + att_b2

    iota2 = jax.lax.broadcasted_iota(jnp.int32, (BB, L), 1)
    mask2 = iota2 < lens
    s = jnp.where(mask2, s, -1e9)
    smax = jnp.max(s, axis=-1, keepdims=True)
    e = jnp.exp(s - smax)
    w = e / jnp.sum(e, axis=-1, keepdims=True)
    context = jnp.sum(w[:, :, None] * h3, axis=1)
    denom = jnp.maximum(jnp.sum(mask2.astype(f32), axis=-1, keepdims=True),
                        1.0)
    summary = jnp.sum(h3, axis=1) / denom

    a1 = scal_ref[0:1, 2:3]
    a2 = scal_ref[0:1, 3:4]
    b3 = scal_ref[0:1, 4:5]
    hw1 = hw1_ref[...]
    pieces = (cand, context, summary, user, dense, cand * context,
              jnp.abs(cand - user))
    acc = dot(pieces[0], hw1[0:H])
    for kk in range(1, 7):
        acc = acc + dot(pieces[kk], hw1[kk * H:(kk + 1) * H])
    h1 = acc + hb1_ref[...]
    h1 = jnp.where(h1 >= 0, h1, h1 * a1)
    h2 = dot(h1, hw2_ref[...]) + hb2_ref[...]
    h2 = jnp.where(h2 >= 0, h2, h2 * a2)
    out_ref[...] = jnp.sum(h2 * hw3_ref[...], axis=-1, keepdims=True) + b3


def _fold_weights(p):
    tok_w = p['tok_W']
    tok_b = p['tok_b']

    def pack_ln(bias, g, beta):
        z = jnp.zeros((8, H), jnp.float32)
        return z.at[0].set(bias).at[1].set(g).at[2].set(beta)

    def fold(w_big, n):
        blocks = [jnp.tile(tok_w @ w_big[k * H:(k + 1) * H], (2, 1))
                  for k in range(n)]
        bias = sum(tok_b @ w_big[k * H:(k + 1) * H] for k in range(n))
        return jnp.concatenate(blocks, axis=0), bias

    whist, bh = fold(p['hist_W'], 3)
    fuser, bu = fold(p['user_W'], 2)
    fcand, bc = fold(p['cand_W'], 3)

    w1 = p['att_W1']
    w1ac = w1[0:H] + w1[2 * H:3 * H]
    w1h = w1[H:2 * H] - w1[2 * H:3 * H]
    w1p = w1[3 * H:4 * H]
    attp = jnp.zeros((8, H), jnp.float32)
    attp = attp.at[0].set(p['att_b1']).at[1].set(p['att_W2'].reshape(H))
    scal = jnp.concatenate([
        jnp.stack([p['att_a'], p['att_b2'][0], p['head_a1'], p['head_a2'],
                   p['head_b3'][0]]),
        jnp.zeros((3,), jnp.float32)]).reshape(1, 8)

    return dict(
        whist=whist,
        lnh=pack_ln(p['hist_b'] + bh, p['hist_g'], p['hist_beta']),
        fuser=fuser,
        lnu=pack_ln(p['user_b'] + bu, p['user_g'], p['user_beta']),
        fcand=fcand,
        lnc=pack_ln(p['cand_b'] + bc, p['cand_g'], p['cand_beta']),
        wdense=p['dense_W'],
        lnd=pack_ln(p['dense_b'], p['dense_g'], p['dense_beta']),
        w1ac=w1ac, w1h=w1h, w1p=w1p, attp=attp,
        hw1=p['head_W1'], hb1=p['head_b1'].reshape(1, HEAD_H),
        hw2=p['head_W2'], hb2=p['head_b2'].reshape(1, MLP_H),
        hw3=p['head_W3'].reshape(1, MLP_H),
        scal=scal,
    )


def _dense_forward(embs, tok_idx2d, lengths, pooled, sp_idx2d,
                   dense_features, p):
    fw = _fold_weights(p)
    lens2 = lengths.astype(jnp.int32).reshape(B, 1)

    def full(shp):
        return pl.BlockSpec(shp, lambda i: tuple(0 for _ in shp))

    out = pl.pallas_call(
        _dense_kernel,
        grid=(B // BB,),
        in_specs=[
            pl.BlockSpec((BB * L, DP), lambda i: (i, 0)),
            pl.BlockSpec((BB * L, DP), lambda i: (i, 0)),
            pl.BlockSpec((BB * L, DP), lambda i: (i, 0)),
            pl.BlockSpec((BB, L), lambda i: (i, 0)),
            pl.BlockSpec((BB, L), lambda i: (i, 0)),
            pl.BlockSpec((BB, L), lambda i: (i, 0)),
            pl.BlockSpec((BB, 1), lambda i: (i, 0)),
            pl.BlockSpec((BB * POOL, DP), lambda i: (i, 0)),
            pl.BlockSpec((BB * POOL, DP), lambda i: (i, 0)),
            pl.BlockSpec((BB * POOL, DP), lambda i: (i, 0)),
            pl.BlockSpec((BB * POOL, DP), lambda i: (i, 0)),
            pl.BlockSpec((BB * POOL, DP), lambda i: (i, 0)),
            pl.BlockSpec((BB, POOL), lambda i: (i, 0)),
            pl.BlockSpec((BB, POOL), lambda i: (i, 0)),
            pl.BlockSpec((BB, POOL), lambda i: (i, 0)),
            pl.BlockSpec((BB, POOL), lambda i: (i, 0)),
            pl.BlockSpec((BB, POOL), lambda i: (i, 0)),
            pl.BlockSpec((BB, DENSE_DIM), lambda i: (i, 0)),
            full((3 * DP, H)), full((8, H)), full((2 * DP, H)), full((8, H)),
            full((3 * DP, H)), full((8, H)), full((DENSE_DIM, H)),
            full((8, H)), full((H, H)), full((H, H)), full((H, H)),
            full((8, H)), full((7 * H, HEAD_H)), full((1, HEAD_H)),
            full((HEAD_H, MLP_H)), full((1, MLP_H)), full((1, MLP_H)),
            full((1, 8)),
        ],
        out_specs=pl.BlockSpec((BB, 1), lambda i: (i, 0)),
        out_shape=jax.ShapeDtypeStruct((B, 1), jnp.float32),
    )(*embs, *tok_idx2d, lens2, *pooled, *sp_idx2d, dense_features,
      fw['whist'].astype(jnp.bfloat16), fw['lnh'], fw['fuser'], fw['lnu'],
      fw['fcand'], fw['lnc'], fw['wdense'], fw['lnd'], fw['w1ac'],
      fw['w1h'].astype(jnp.bfloat16), fw['w1p'].astype(jnp.bfloat16),
      fw['attp'],
      fw['hw1'], fw['hb1'], fw['hw2'], fw['hb2'], fw['hw3'], fw['scal'])
    return out.reshape(B)


def kernel(history_post_tokens, history_author_tokens, history_action_tokens,
           history_lengths, user_tokens_idx, context_tokens_idx,
           candidate_tokens_idx, candidate_post_tokens_idx,
           candidate_author_tokens_idx, dense_features, params):
    p = params
    tok_idx2d = [t.astype(jnp.int32)
                 for t in (history_post_tokens, history_author_tokens,
                           history_action_tokens)]
    tok_half = [jnp.right_shift(t.reshape(B * L), 1) for t in tok_idx2d]
    sp_idx_map = {
        'user_tokens': user_tokens_idx,
        'context_tokens': context_tokens_idx,
        'candidate_post_tokens': candidate_post_tokens_idx,
        'candidate_author_tokens': candidate_author_tokens_idx,
        'candidate_tokens': candidate_tokens_idx,
    }
    sp_idx2d = [sp_idx_map[n].astype(jnp.int32) for n in _SP_NAMES]
    sp_half = [jnp.right_shift(t.reshape(B * POOL), 1) for t in sp_idx2d]

    tok_pairs = p['token_table'].reshape(-1, DP)
    sp_pairs = [p[n + '_table'].reshape(-1, DP) for n in _SP_NAMES]

    gathered = _sc_gather(tok_pairs, tok_half, sp_pairs, sp_half)
    embs = gathered[0:3]
    pooled = gathered[3:]
    return _dense_forward(embs, tok_idx2d, history_lengths, pooled, sp_idx2d,
                          dense_features, p)
